# baseline (speedup 1.0000x reference)
import functools

import jax
import jax.numpy as jnp
from jax import lax
from jax.experimental import pallas as pl
from jax.experimental.pallas import tpu as pltpu
from jax.experimental.pallas import tpu_sc as plsc

N = 10000
E = 320000
NW = 32
PERW = E // NW
CH = 2000
NCH = PERW // CH

EB = 2560
EG = E // EB


def _sc_gather(table, idx):
    mesh = plsc.VectorSubcoreMesh(core_axis_name="c", subcore_axis_name="s")

    @functools.partial(
        pl.kernel, mesh=mesh,
        out_type=jax.ShapeDtypeStruct((E, 16), jnp.float32),
        compiler_params=pltpu.CompilerParams(use_tc_tiling_on_sc=False),
        scratch_types=[
            pltpu.VMEM((CH,), jnp.int32),
            pltpu.VMEM((CH, 16), jnp.float32),
            pltpu.SemaphoreType.DMA,
        ],
    )
    def gather_k(table_hbm, idx_hbm, out_hbm, idx_v, rows_v, sem):
        wid = lax.axis_index("s") * 2 + lax.axis_index("c")
        base = wid * PERW

        def body(ci, carry):
            off = base + ci * CH
            pltpu.sync_copy(idx_hbm.at[pl.ds(off, CH)], idx_v)
            pltpu.async_copy(table_hbm.at[idx_v], rows_v, sem).wait()
            pltpu.sync_copy(rows_v, out_hbm.at[pl.ds(off, CH)])
            return carry

        lax.fori_loop(0, NCH, body, 0)

    return gather_k(table, idx)


def _sc_scatter_add(vals, idx, zeros):
    mesh = plsc.VectorSubcoreMesh(core_axis_name="c", subcore_axis_name="s")

    @functools.partial(
        pl.kernel, mesh=mesh,
        out_type=jax.ShapeDtypeStruct((2, N, 16), jnp.float32),
        compiler_params=pltpu.CompilerParams(use_tc_tiling_on_sc=False),
        scratch_types=[
            pltpu.VMEM((CH,), jnp.int32),
            pltpu.VMEM((CH, 16), jnp.float32),
            pltpu.VMEM_SHARED((N, 16), jnp.float32),
        ],
    )
    def scatter_k(vals_hbm, idx_hbm, zeros_hbm, out_hbm, idx_v, val_v, acc_sh):
        cid = lax.axis_index("c")
        sid = lax.axis_index("s")

        @pl.when(sid == 0)
        def _init():
            pltpu.sync_copy(zeros_hbm, acc_sh)

        plsc.subcore_barrier()

        base = (sid * 2 + cid) * PERW

        def body(ci, carry):
            off = base + ci * CH
            pltpu.sync_copy(idx_hbm.at[pl.ds(off, CH)], idx_v)
            pltpu.sync_copy(vals_hbm.at[pl.ds(off, CH)], val_v)
            pltpu.sync_copy(val_v, acc_sh.at[idx_v], add=True)
            return carry

        lax.fori_loop(0, NCH, body, 0)
        plsc.subcore_barrier()

        rows = N // 16
        pltpu.sync_copy(acc_sh.at[pl.ds(sid * rows, rows)],
                        out_hbm.at[cid, pl.ds(sid * rows, rows)])

    return scatter_k(vals, idx, zeros)


def _edges_body(ea_ref, xj_ref, w1_ref, b1_ref, w2_ref, b2_ref, t_ref, r_ref,
                out_ref):
    q = EB // 8
    xj_p = xj_ref[...]
    ea = ea_ref[...]
    xj = jnp.concatenate([xj_p[:, 16 * j:16 * (j + 1)] for j in range(8)],
                         axis=0)
    h = jnp.dot(ea, w1_ref[...], preferred_element_type=jnp.float32)
    h = jnp.maximum(h + b1_ref[...], 0.0)
    wm = jnp.dot(h.astype(jnp.bfloat16), w2_ref[...],
                 preferred_element_type=jnp.float32)
    wm = wm + b2_ref[...]
    xt = jnp.dot(xj, t_ref[...], preferred_element_type=jnp.float32)
    msg = jnp.dot(xt * wm, r_ref[...], preferred_element_type=jnp.float32)
    out_ref[...] = jnp.concatenate([msg[q * j:q * (j + 1), :]
                                    for j in range(8)], axis=1)


def _tc_edges(ea, xj_p, w1, b1, w2, b2, t, r):
    h1 = w1.shape[1]
    o1 = w2.shape[1]
    grid = (EG,)
    return pl.pallas_call(
        _edges_body,
        grid=grid,
        in_specs=[
            pl.BlockSpec((EB, 16), lambda i: (i, 0)),
            pl.BlockSpec((EB // 8, 128), lambda i: (i, 0)),
            pl.BlockSpec((16, h1), lambda i: (0, 0)),
            pl.BlockSpec((1, h1), lambda i: (0, 0)),
            pl.BlockSpec((h1, o1), lambda i: (0, 0)),
            pl.BlockSpec((1, o1), lambda i: (0, 0)),
            pl.BlockSpec((16, o1), lambda i: (0, 0)),
            pl.BlockSpec((o1, 16), lambda i: (0, 0)),
        ],
        out_specs=pl.BlockSpec((EB // 8, 128), lambda i: (i, 0)),
        out_shape=jax.ShapeDtypeStruct((E // 8, 128), jnp.float32),
    )(ea, xj_p, w1, b1, w2, b2, t, r)


def _comb1_body(p_ref, x_ref, root_ref, bias_ref, out_ref):
    agg = p_ref[0] + p_ref[1]
    rt = jnp.dot(x_ref[...], root_ref[...], preferred_element_type=jnp.float32)
    out_ref[...] = jnp.maximum(agg + rt + bias_ref[...], 0.0)


def _tc_combine1(parts, x, root, bias):
    return pl.pallas_call(
        _comb1_body,
        out_shape=jax.ShapeDtypeStruct((N, 16), jnp.float32),
    )(parts, x, root, bias)


def _comb2_body(p_ref, x_ref, root_ref, bias_ref, out_ref):
    agg = p_ref[0] + p_ref[1]
    rt = jnp.dot(x_ref[...], root_ref[...], preferred_element_type=jnp.float32)
    y = agg[:, :8] + rt + bias_ref[...]
    m = jnp.max(y, axis=1, keepdims=True)
    lse = jnp.log(jnp.sum(jnp.exp(y - m), axis=1, keepdims=True)) + m
    out_ref[...] = y - lse


def _tc_combine2(parts, x, root, bias):
    return pl.pallas_call(
        _comb2_body,
        out_shape=jax.ShapeDtypeStruct((N, 8), jnp.float32),
    )(parts, x, root, bias)


def kernel(x_in, edge_index, edge_atts, w11, b11, w12, b12, root1, bias1,
           w21, b21, w22, b22, root2, bias2):
    src = edge_index[0].reshape(EG, 8, EB // 8).transpose(0, 2, 1).reshape(E)
    dst = edge_index[1].reshape(EG, 8, EB // 8).transpose(0, 2, 1).reshape(E)

    f32 = jnp.float32
    eye16 = jnp.eye(16, dtype=f32)
    t1 = jnp.repeat(eye16, 16, axis=1)
    r1 = jnp.tile(eye16, (16, 1))
    t2 = jnp.repeat(eye16, 8, axis=1)
    r2 = jnp.pad(jnp.tile(jnp.eye(8, dtype=f32), (16, 1)), ((0, 0), (0, 8)))

    zeros = jnp.zeros((N, 16), f32)

    xj1 = _sc_gather(x_in, src)
    msg1 = _tc_edges(edge_atts, xj1.reshape(E // 8, 128), w11, b11[None, :],
                     w12.astype(jnp.bfloat16), b12[None, :], t1, r1)
    p1 = _sc_scatter_add(msg1.reshape(E, 16), dst, zeros)
    x1 = _tc_combine1(p1, x_in, root1, bias1[None, :])

    xj2 = _sc_gather(x1, src)
    msg2 = _tc_edges(edge_atts, xj2.reshape(E // 8, 128), w21, b21[None, :],
                     w22.astype(jnp.bfloat16), b22[None, :], t2, r2)
    p2 = _sc_scatter_add(msg2.reshape(E, 16), dst, zeros)
    return _tc_combine2(p2, x1, root2, bias2[None, :])


# speedup vs baseline: 1.0573x; 1.0573x over previous
import functools

import jax
import jax.numpy as jnp
from jax import lax
from jax.experimental import pallas as pl
from jax.experimental.pallas import tpu as pltpu
from jax.experimental.pallas import tpu_sc as plsc

N = 10000
E = 320000
NW = 32
PERW = E // NW
CH = 2000
NCH = PERW // CH

EB = 2560
EG = E // EB


def _sc_gather(table, idx):
    mesh = plsc.VectorSubcoreMesh(core_axis_name="c", subcore_axis_name="s")

    @functools.partial(
        pl.kernel, mesh=mesh,
        out_type=jax.ShapeDtypeStruct((E, 16), jnp.float32),
        compiler_params=pltpu.CompilerParams(use_tc_tiling_on_sc=False),
        scratch_types=[
            pltpu.VMEM((CH,), jnp.int32),
            pltpu.VMEM((CH, 16), jnp.float32),
            pltpu.SemaphoreType.DMA,
        ],
    )
    def gather_k(table_hbm, idx_hbm, out_hbm, idx_v, rows_v, sem):
        wid = lax.axis_index("s") * 2 + lax.axis_index("c")
        base = wid * PERW

        def body(ci, carry):
            off = base + ci * CH
            pltpu.sync_copy(idx_hbm.at[pl.ds(off, CH)], idx_v)
            pltpu.async_copy(table_hbm.at[idx_v], rows_v, sem).wait()
            pltpu.sync_copy(rows_v, out_hbm.at[pl.ds(off, CH)])
            return carry

        lax.fori_loop(0, NCH, body, 0)

    return gather_k(table, idx)


def _sc_scatter_add(vals, idx, zeros):
    mesh = plsc.VectorSubcoreMesh(core_axis_name="c", subcore_axis_name="s")

    @functools.partial(
        pl.kernel, mesh=mesh,
        out_type=jax.ShapeDtypeStruct((2, N, 16), jnp.float32),
        compiler_params=pltpu.CompilerParams(use_tc_tiling_on_sc=False),
        scratch_types=[
            pltpu.VMEM((CH,), jnp.int32),
            pltpu.VMEM((CH, 16), jnp.float32),
            pltpu.VMEM_SHARED((N, 16), jnp.float32),
        ],
    )
    def scatter_k(vals_hbm, idx_hbm, zeros_hbm, out_hbm, idx_v, val_v, acc_sh):
        cid = lax.axis_index("c")
        sid = lax.axis_index("s")

        @pl.when(sid == 0)
        def _init():
            pltpu.sync_copy(zeros_hbm, acc_sh)

        plsc.subcore_barrier()

        base = (sid * 2 + cid) * PERW

        def body(ci, carry):
            off = base + ci * CH
            pltpu.sync_copy(idx_hbm.at[pl.ds(off, CH)], idx_v)
            pltpu.sync_copy(vals_hbm.at[pl.ds(off, CH)], val_v)
            pltpu.sync_copy(val_v, acc_sh.at[idx_v], add=True)
            return carry

        lax.fori_loop(0, NCH, body, 0)
        plsc.subcore_barrier()

        rows = N // 16
        pltpu.sync_copy(acc_sh.at[pl.ds(sid * rows, rows)],
                        out_hbm.at[cid, pl.ds(sid * rows, rows)])

    return scatter_k(vals, idx, zeros)


def _edges_body(ea_ref, xj_ref, w1_ref, b1_ref, w2_ref, b2_ref, t_ref, r_ref,
                out_ref):
    q = EB // 8
    ea = ea_ref[...].reshape(EB, 16)
    xj_p = xj_ref[...]
    xj = jnp.concatenate([xj_p[:, 16 * j:16 * (j + 1)] for j in range(8)],
                         axis=0)
    h = jnp.dot(ea, w1_ref[...], preferred_element_type=jnp.float32)
    h = jnp.maximum(h + b1_ref[...], 0.0)
    wm = jnp.dot(h, w2_ref[...], preferred_element_type=jnp.float32)
    wm = wm + b2_ref[...]
    xt = jnp.dot(xj, t_ref[...], preferred_element_type=jnp.float32)
    msg = jnp.dot(xt * wm, r_ref[...], preferred_element_type=jnp.float32)
    out_ref[...] = jnp.concatenate([msg[q * j:q * (j + 1), :]
                                    for j in range(8)], axis=1)


def _tc_edges(ea, xj_p, w1, b1, w2, b2, t, r):
    h1 = w1.shape[1]
    o1 = w2.shape[1]
    grid = (EG,)
    return pl.pallas_call(
        _edges_body,
        grid=grid,
        in_specs=[
            pl.BlockSpec((EB // 8, 8, 16), lambda i: (i, 0, 0)),
            pl.BlockSpec((EB // 8, 128), lambda i: (i, 0)),
            pl.BlockSpec((16, h1), lambda i: (0, 0)),
            pl.BlockSpec((1, h1), lambda i: (0, 0)),
            pl.BlockSpec((h1, o1), lambda i: (0, 0)),
            pl.BlockSpec((1, o1), lambda i: (0, 0)),
            pl.BlockSpec((16, o1), lambda i: (0, 0)),
            pl.BlockSpec((o1, 16), lambda i: (0, 0)),
        ],
        out_specs=pl.BlockSpec((EB // 8, 128), lambda i: (i, 0)),
        out_shape=jax.ShapeDtypeStruct((E // 8, 128), jnp.float32),
    )(ea, xj_p, w1, b1, w2, b2, t, r)


def _comb1_body(p_ref, x_ref, root_ref, bias_ref, out_ref):
    agg = p_ref[0] + p_ref[1]
    rt = jnp.dot(x_ref[...], root_ref[...], preferred_element_type=jnp.float32)
    out_ref[...] = jnp.maximum(agg + rt + bias_ref[...], 0.0)


def _tc_combine1(parts, x, root, bias):
    return pl.pallas_call(
        _comb1_body,
        out_shape=jax.ShapeDtypeStruct((N, 16), jnp.float32),
    )(parts, x, root, bias)


def _comb2_body(p_ref, x_ref, root_ref, bias_ref, out_ref):
    agg = p_ref[0] + p_ref[1]
    rt = jnp.dot(x_ref[...], root_ref[...], preferred_element_type=jnp.float32)
    y = agg[:, :8] + rt + bias_ref[...]
    m = jnp.max(y, axis=1, keepdims=True)
    lse = jnp.log(jnp.sum(jnp.exp(y - m), axis=1, keepdims=True)) + m
    out_ref[...] = y - lse


def _tc_combine2(parts, x, root, bias):
    return pl.pallas_call(
        _comb2_body,
        out_shape=jax.ShapeDtypeStruct((N, 8), jnp.float32),
    )(parts, x, root, bias)


def kernel(x_in, edge_index, edge_atts, w11, b11, w12, b12, root1, bias1,
           w21, b21, w22, b22, root2, bias2):
    ei_p = edge_index.reshape(2, EG, 8, EB // 8).transpose(0, 1, 3, 2)
    ei_p = ei_p.reshape(2, E)
    src = ei_p[0]
    dst = ei_p[1]

    f32 = jnp.float32
    eye16 = jnp.eye(16, dtype=f32)
    t1 = jnp.repeat(eye16, 16, axis=1)
    r1 = jnp.tile(eye16, (16, 1))
    t2 = jnp.repeat(eye16, 8, axis=1)
    r2 = jnp.pad(jnp.tile(jnp.eye(8, dtype=f32), (16, 1)), ((0, 0), (0, 8)))

    zeros = jnp.zeros((N, 16), f32)
    ea_p = edge_atts.reshape(E // 8, 8, 16)

    xj1 = _sc_gather(x_in, src)
    msg1 = _tc_edges(ea_p, xj1.reshape(E // 8, 128), w11, b11[None, :],
                     w12, b12[None, :], t1, r1)
    p1 = _sc_scatter_add(msg1.reshape(E, 16), dst, zeros)
    x1 = _tc_combine1(p1, x_in, root1, bias1[None, :])

    xj2 = _sc_gather(x1, src)
    msg2 = _tc_edges(ea_p, xj2.reshape(E // 8, 128), w21, b21[None, :],
                     w22, b22[None, :], t2, r2)
    p2 = _sc_scatter_add(msg2.reshape(E, 16), dst, zeros)
    return _tc_combine2(p2, x1, root2, bias2[None, :])


# speedup vs baseline: 1.1238x; 1.0629x over previous
import functools

import jax
import jax.numpy as jnp
from jax import lax
from jax.experimental import pallas as pl
from jax.experimental.pallas import tpu as pltpu
from jax.experimental.pallas import tpu_sc as plsc

N = 10000
E = 320000
NW = 32
PERW = E // NW
CH = 2000
NCH = PERW // CH

EB = 2560
EG = E // EB


def _sc_gather(table, idx, opos):
    mesh = plsc.VectorSubcoreMesh(core_axis_name="c", subcore_axis_name="s")

    @functools.partial(
        pl.kernel, mesh=mesh,
        out_type=jax.ShapeDtypeStruct((E, 16), jnp.float32),
        compiler_params=pltpu.CompilerParams(use_tc_tiling_on_sc=False),
        scratch_types=[
            pltpu.VMEM((CH,), jnp.int32),
            pltpu.VMEM((CH,), jnp.int32),
            pltpu.VMEM((CH, 16), jnp.float32),
            pltpu.SemaphoreType.DMA,
        ],
    )
    def gather_k(table_hbm, idx_hbm, opos_hbm, out_hbm, idx_v, pos_v, rows_v,
                 sem):
        wid = lax.axis_index("s") * 2 + lax.axis_index("c")
        base = wid * PERW

        def body(ci, carry):
            off = base + ci * CH
            pltpu.sync_copy(idx_hbm.at[pl.ds(off, CH)], idx_v)
            pltpu.sync_copy(opos_hbm.at[pl.ds(off, CH)], pos_v)
            pltpu.async_copy(table_hbm.at[idx_v], rows_v, sem).wait()
            pltpu.async_copy(rows_v, out_hbm.at[pos_v], sem).wait()
            return carry

        lax.fori_loop(0, NCH, body, 0)

    return gather_k(table, idx, opos)


def _sc_scatter_add(vals, idx, vpos, zeros):
    mesh = plsc.VectorSubcoreMesh(core_axis_name="c", subcore_axis_name="s")

    @functools.partial(
        pl.kernel, mesh=mesh,
        out_type=jax.ShapeDtypeStruct((2, N, 16), jnp.float32),
        compiler_params=pltpu.CompilerParams(use_tc_tiling_on_sc=False),
        scratch_types=[
            pltpu.VMEM((CH,), jnp.int32),
            pltpu.VMEM((CH,), jnp.int32),
            pltpu.VMEM((CH, 16), jnp.float32),
            pltpu.VMEM_SHARED((N, 16), jnp.float32),
            pltpu.SemaphoreType.DMA,
        ],
    )
    def scatter_k(vals_hbm, idx_hbm, vpos_hbm, zeros_hbm, out_hbm, idx_v,
                  pos_v, val_v, acc_sh, sem):
        cid = lax.axis_index("c")
        sid = lax.axis_index("s")

        @pl.when(sid == 0)
        def _init():
            pltpu.sync_copy(zeros_hbm, acc_sh)

        plsc.subcore_barrier()

        base = (sid * 2 + cid) * PERW

        def body(ci, carry):
            off = base + ci * CH
            pltpu.sync_copy(idx_hbm.at[pl.ds(off, CH)], idx_v)
            pltpu.sync_copy(vpos_hbm.at[pl.ds(off, CH)], pos_v)
            pltpu.async_copy(vals_hbm.at[pos_v], val_v, sem).wait()
            pltpu.sync_copy(val_v, acc_sh.at[idx_v], add=True)
            return carry

        lax.fori_loop(0, NCH, body, 0)
        plsc.subcore_barrier()

        rows = N // 16
        pltpu.sync_copy(acc_sh.at[pl.ds(sid * rows, rows)],
                        out_hbm.at[cid, pl.ds(sid * rows, rows)])

    return scatter_k(vals, idx, vpos, zeros)


def _edges_body(ea_ref, xj_ref, w1_ref, b1_ref, w2_ref, b2_ref, t_ref, r_ref,
                out_ref):
    q = EB // 8
    ea = ea_ref[...].reshape(EB, 16)
    xj_p = xj_ref[...]
    xj = jnp.concatenate([xj_p[:, 16 * j:16 * (j + 1)] for j in range(8)],
                         axis=0)
    h = jnp.dot(ea, w1_ref[...], preferred_element_type=jnp.float32)
    h = jnp.maximum(h + b1_ref[...], 0.0)
    wm = jnp.dot(h, w2_ref[...], preferred_element_type=jnp.float32)
    wm = wm + b2_ref[...]
    xt = jnp.dot(xj, t_ref[...], preferred_element_type=jnp.float32)
    msg = jnp.dot(xt * wm, r_ref[...], preferred_element_type=jnp.float32)
    out_ref[...] = jnp.concatenate([msg[q * j:q * (j + 1), :]
                                    for j in range(8)], axis=1)


def _tc_edges(ea, xj_p, w1, b1, w2, b2, t, r):
    h1 = w1.shape[1]
    o1 = w2.shape[1]
    grid = (EG,)
    return pl.pallas_call(
        _edges_body,
        grid=grid,
        in_specs=[
            pl.BlockSpec((EB // 8, 8, 16), lambda i: (i, 0, 0)),
            pl.BlockSpec((EB // 8, 128), lambda i: (i, 0)),
            pl.BlockSpec((16, h1), lambda i: (0, 0)),
            pl.BlockSpec((1, h1), lambda i: (0, 0)),
            pl.BlockSpec((h1, o1), lambda i: (0, 0)),
            pl.BlockSpec((1, o1), lambda i: (0, 0)),
            pl.BlockSpec((16, o1), lambda i: (0, 0)),
            pl.BlockSpec((o1, 16), lambda i: (0, 0)),
        ],
        out_specs=pl.BlockSpec((EB // 8, 128), lambda i: (i, 0)),
        out_shape=jax.ShapeDtypeStruct((E // 8, 128), jnp.float32),
    )(ea, xj_p, w1, b1, w2, b2, t, r)


def _comb1_body(p_ref, x_ref, root_ref, bias_ref, out_ref):
    agg = p_ref[0] + p_ref[1]
    rt = jnp.dot(x_ref[...], root_ref[...], preferred_element_type=jnp.float32)
    out_ref[...] = jnp.maximum(agg + rt + bias_ref[...], 0.0)


def _tc_combine1(parts, x, root, bias):
    return pl.pallas_call(
        _comb1_body,
        out_shape=jax.ShapeDtypeStruct((N, 16), jnp.float32),
    )(parts, x, root, bias)


def _comb2_body(p_ref, x_ref, root_ref, bias_ref, out_ref):
    agg = p_ref[0] + p_ref[1]
    rt = jnp.dot(x_ref[...], root_ref[...], preferred_element_type=jnp.float32)
    y = agg[:, :8] + rt + bias_ref[...]
    m = jnp.max(y, axis=1, keepdims=True)
    lse = jnp.log(jnp.sum(jnp.exp(y - m), axis=1, keepdims=True)) + m
    out_ref[...] = y - lse


def _tc_combine2(parts, x, root, bias):
    return pl.pallas_call(
        _comb2_body,
        out_shape=jax.ShapeDtypeStruct((N, 8), jnp.float32),
    )(parts, x, root, bias)


def kernel(x_in, edge_index, edge_atts, w11, b11, w12, b12, root1, bias1,
           w21, b21, w22, b22, root2, bias2):
    src = edge_index[0]
    dst = edge_index[1]
    qfull = jnp.arange(E, dtype=jnp.int32).reshape(EG, EB // 8, 8)
    qfull = qfull.transpose(0, 2, 1).reshape(E)

    f32 = jnp.float32
    eye16 = jnp.eye(16, dtype=f32)
    t1 = jnp.repeat(eye16, 16, axis=1)
    r1 = jnp.tile(eye16, (16, 1))
    t2 = jnp.repeat(eye16, 8, axis=1)
    r2 = jnp.pad(jnp.tile(jnp.eye(8, dtype=f32), (16, 1)), ((0, 0), (0, 8)))

    zeros = jnp.zeros((N, 16), f32)
    ea_p = edge_atts.reshape(E // 8, 8, 16)

    xj1 = _sc_gather(x_in, src, qfull)
    msg1 = _tc_edges(ea_p, xj1.reshape(E // 8, 128), w11, b11[None, :],
                     w12, b12[None, :], t1, r1)
    p1 = _sc_scatter_add(msg1.reshape(E, 16), dst, qfull, zeros)
    x1 = _tc_combine1(p1, x_in, root1, bias1[None, :])

    xj2 = _sc_gather(x1, src, qfull)
    msg2 = _tc_edges(ea_p, xj2.reshape(E // 8, 128), w21, b21[None, :],
                     w22, b22[None, :], t2, r2)
    p2 = _sc_scatter_add(msg2.reshape(E, 16), dst, qfull, zeros)
    return _tc_combine2(p2, x1, root2, bias2[None, :])


# speedup vs baseline: 1.1611x; 1.0332x over previous
import functools

import jax
import jax.numpy as jnp
from jax import lax
from jax.experimental import pallas as pl
from jax.experimental.pallas import tpu as pltpu
from jax.experimental.pallas import tpu_sc as plsc

N = 10000
E = 320000
NW = 32
PERW = E // NW
CH = 2000
NCH = PERW // CH

EB = 2560
EG = E // EB


def _sc_gather(table, idx, opos):
    mesh = plsc.VectorSubcoreMesh(core_axis_name="c", subcore_axis_name="s")

    @functools.partial(
        pl.kernel, mesh=mesh,
        out_type=jax.ShapeDtypeStruct((E, 16), jnp.float32),
        compiler_params=pltpu.CompilerParams(use_tc_tiling_on_sc=False),
        scratch_types=[
            pltpu.VMEM((CH,), jnp.int32),
            pltpu.VMEM((CH,), jnp.int32),
            pltpu.VMEM((CH, 16), jnp.float32),
            pltpu.SemaphoreType.DMA,
        ],
    )
    def gather_k(table_hbm, idx_hbm, opos_hbm, out_hbm, idx_v, pos_v, rows_v,
                 sem):
        wid = lax.axis_index("s") * 2 + lax.axis_index("c")
        base = wid * PERW

        def body(ci, carry):
            off = base + ci * CH
            pltpu.sync_copy(idx_hbm.at[pl.ds(off, CH)], idx_v)
            pltpu.sync_copy(opos_hbm.at[pl.ds(off, CH)], pos_v)
            pltpu.async_copy(table_hbm.at[idx_v], rows_v, sem).wait()
            pltpu.async_copy(rows_v, out_hbm.at[pos_v], sem).wait()
            return carry

        lax.fori_loop(0, NCH, body, 0)

    return gather_k(table, idx, opos)


def _sc_scatter_add(vals, idx, vpos, zeros):
    mesh = plsc.VectorSubcoreMesh(core_axis_name="c", subcore_axis_name="s")

    @functools.partial(
        pl.kernel, mesh=mesh,
        out_type=jax.ShapeDtypeStruct((2, N, 16), jnp.float32),
        compiler_params=pltpu.CompilerParams(use_tc_tiling_on_sc=False),
        scratch_types=[
            pltpu.VMEM((CH,), jnp.int32),
            pltpu.VMEM((CH,), jnp.int32),
            pltpu.VMEM((CH, 16), jnp.float32),
            pltpu.VMEM_SHARED((N, 16), jnp.float32),
            pltpu.SemaphoreType.DMA,
        ],
    )
    def scatter_k(vals_hbm, idx_hbm, vpos_hbm, zeros_hbm, out_hbm, idx_v,
                  pos_v, val_v, acc_sh, sem):
        cid = lax.axis_index("c")
        sid = lax.axis_index("s")

        @pl.when(sid == 0)
        def _init():
            pltpu.sync_copy(zeros_hbm, acc_sh)

        plsc.subcore_barrier()

        base = (sid * 2 + cid) * PERW

        def body(ci, carry):
            off = base + ci * CH
            pltpu.sync_copy(idx_hbm.at[pl.ds(off, CH)], idx_v)
            pltpu.sync_copy(vpos_hbm.at[pl.ds(off, CH)], pos_v)
            pltpu.async_copy(vals_hbm.at[pos_v], val_v, sem).wait()
            pltpu.sync_copy(val_v, acc_sh.at[idx_v], add=True)
            return carry

        lax.fori_loop(0, NCH, body, 0)
        plsc.subcore_barrier()

        rows = N // 16
        pltpu.sync_copy(acc_sh.at[pl.ds(sid * rows, rows)],
                        out_hbm.at[cid, pl.ds(sid * rows, rows)])

    return scatter_k(vals, idx, vpos, zeros)


def _unpack(p):
    return jnp.concatenate([p[:, 16 * j:16 * (j + 1)] for j in range(8)],
                           axis=0)


def _pack(x):
    q = EB // 8
    return jnp.concatenate([x[q * j:q * (j + 1), :] for j in range(8)],
                           axis=1)


def _edges_math(ea, xj, w1_ref, b1_ref, w2_ref, b2_ref, t_ref, r_ref):
    h = jnp.dot(ea, w1_ref[...], preferred_element_type=jnp.float32)
    h = jnp.maximum(h + b1_ref[...], 0.0)
    wm = jnp.dot(h, w2_ref[...], preferred_element_type=jnp.float32)
    wm = wm + b2_ref[...]
    xt = jnp.dot(xj, t_ref[...], preferred_element_type=jnp.float32)
    return jnp.dot(xt * wm, r_ref[...], preferred_element_type=jnp.float32)


def _edges1_body(ea_ref, xj_ref, w1_ref, b1_ref, w2_ref, b2_ref, t_ref,
                 r_ref, out_ref, eap_ref):
    ea = ea_ref[...].reshape(EB, 16)
    xj = _unpack(xj_ref[...])
    msg = _edges_math(ea, xj, w1_ref, b1_ref, w2_ref, b2_ref, t_ref, r_ref)
    out_ref[...] = _pack(msg)
    eap_ref[...] = _pack(ea)


def _edges2_body(ea_ref, xj_ref, w1_ref, b1_ref, w2_ref, b2_ref, t_ref,
                 r_ref, out_ref):
    ea = _unpack(ea_ref[...])
    xj = _unpack(xj_ref[...])
    msg = _edges_math(ea, xj, w1_ref, b1_ref, w2_ref, b2_ref, t_ref, r_ref)
    out_ref[...] = _pack(msg)


def _weight_specs(h1, o1):
    return [
        pl.BlockSpec((16, h1), lambda i: (0, 0)),
        pl.BlockSpec((1, h1), lambda i: (0, 0)),
        pl.BlockSpec((h1, o1), lambda i: (0, 0)),
        pl.BlockSpec((1, o1), lambda i: (0, 0)),
        pl.BlockSpec((16, o1), lambda i: (0, 0)),
        pl.BlockSpec((o1, 16), lambda i: (0, 0)),
    ]


def _tc_edges1(ea, xj_p, w1, b1, w2, b2, t, r):
    return pl.pallas_call(
        _edges1_body,
        grid=(EG,),
        in_specs=[
            pl.BlockSpec((EB // 8, 8, 16), lambda i: (i, 0, 0)),
            pl.BlockSpec((EB // 8, 128), lambda i: (i, 0)),
        ] + _weight_specs(w1.shape[1], w2.shape[1]),
        out_specs=[pl.BlockSpec((EB // 8, 128), lambda i: (i, 0)),
                   pl.BlockSpec((EB // 8, 128), lambda i: (i, 0))],
        out_shape=[jax.ShapeDtypeStruct((E // 8, 128), jnp.float32),
                   jax.ShapeDtypeStruct((E // 8, 128), jnp.float32)],
    )(ea, xj_p, w1, b1, w2, b2, t, r)


def _tc_edges2(ea_p, xj_p, w1, b1, w2, b2, t, r):
    return pl.pallas_call(
        _edges2_body,
        grid=(EG,),
        in_specs=[
            pl.BlockSpec((EB // 8, 128), lambda i: (i, 0)),
            pl.BlockSpec((EB // 8, 128), lambda i: (i, 0)),
        ] + _weight_specs(w1.shape[1], w2.shape[1]),
        out_specs=pl.BlockSpec((EB // 8, 128), lambda i: (i, 0)),
        out_shape=jax.ShapeDtypeStruct((E // 8, 128), jnp.float32),
    )(ea_p, xj_p, w1, b1, w2, b2, t, r)


def _comb1_body(p_ref, x_ref, root_ref, bias_ref, out_ref):
    agg = p_ref[0] + p_ref[1]
    rt = jnp.dot(x_ref[...], root_ref[...], preferred_element_type=jnp.float32)
    out_ref[...] = jnp.maximum(agg + rt + bias_ref[...], 0.0)


def _tc_combine1(parts, x, root, bias):
    return pl.pallas_call(
        _comb1_body,
        out_shape=jax.ShapeDtypeStruct((N, 16), jnp.float32),
    )(parts, x, root, bias)


def _comb2_body(p_ref, x_ref, root_ref, bias_ref, out_ref):
    agg = p_ref[0] + p_ref[1]
    rt = jnp.dot(x_ref[...], root_ref[...], preferred_element_type=jnp.float32)
    y = agg[:, :8] + rt + bias_ref[...]
    m = jnp.max(y, axis=1, keepdims=True)
    lse = jnp.log(jnp.sum(jnp.exp(y - m), axis=1, keepdims=True)) + m
    out_ref[...] = y - lse


def _tc_combine2(parts, x, root, bias):
    return pl.pallas_call(
        _comb2_body,
        out_shape=jax.ShapeDtypeStruct((N, 8), jnp.float32),
    )(parts, x, root, bias)


def kernel(x_in, edge_index, edge_atts, w11, b11, w12, b12, root1, bias1,
           w21, b21, w22, b22, root2, bias2):
    src = edge_index[0]
    dst = edge_index[1]
    qfull = jnp.arange(E, dtype=jnp.int32).reshape(EG, EB // 8, 8)
    qfull = qfull.transpose(0, 2, 1).reshape(E)

    f32 = jnp.float32
    eye16 = jnp.eye(16, dtype=f32)
    t1 = jnp.repeat(eye16, 16, axis=1)
    r1 = jnp.tile(eye16, (16, 1))
    t2 = jnp.repeat(eye16, 8, axis=1)
    r2 = jnp.pad(jnp.tile(jnp.eye(8, dtype=f32), (16, 1)), ((0, 0), (0, 8)))

    zeros = jnp.zeros((N, 16), f32)
    ea_p = edge_atts.reshape(E // 8, 8, 16)

    xj1 = _sc_gather(x_in, src, qfull)
    msg1, ea_relay = _tc_edges1(ea_p, xj1.reshape(E // 8, 128), w11,
                                b11[None, :], w12, b12[None, :], t1, r1)
    p1 = _sc_scatter_add(msg1.reshape(E, 16), dst, qfull, zeros)
    x1 = _tc_combine1(p1, x_in, root1, bias1[None, :])

    xj2 = _sc_gather(x1, src, qfull)
    msg2 = _tc_edges2(ea_relay, xj2.reshape(E // 8, 128), w21, b21[None, :],
                      w22, b22[None, :], t2, r2)
    p2 = _sc_scatter_add(msg2.reshape(E, 16), dst, qfull, zeros)
    return _tc_combine2(p2, x1, root2, bias2[None, :])


# speedup vs baseline: 1.2170x; 1.0482x over previous
import functools

import jax
import jax.numpy as jnp
from jax import lax
from jax.experimental import pallas as pl
from jax.experimental.pallas import tpu as pltpu
from jax.experimental.pallas import tpu_sc as plsc

N = 10000
E = 320000
NW = 32

EB = 2560
EG = E // EB
GA = 64
GB = EG - GA
E_A = GA * EB
E_B = GB * EB


@functools.lru_cache(maxsize=None)
def _sc_gather(etot, ch):
    perw = etot // NW
    nch = perw // ch
    mesh = plsc.VectorSubcoreMesh(core_axis_name="c", subcore_axis_name="s")

    @functools.partial(
        pl.kernel, mesh=mesh,
        out_type=jax.ShapeDtypeStruct((etot, 16), jnp.float32),
        compiler_params=pltpu.CompilerParams(use_tc_tiling_on_sc=False),
        scratch_types=[
            pltpu.VMEM((ch,), jnp.int32),
            pltpu.VMEM((ch,), jnp.int32),
            pltpu.VMEM((ch, 16), jnp.float32),
            pltpu.SemaphoreType.DMA,
        ],
    )
    def gather_k(table_hbm, idx_hbm, opos_hbm, out_hbm, idx_v, pos_v, rows_v,
                 sem):
        wid = lax.axis_index("s") * 2 + lax.axis_index("c")
        base = wid * perw

        def body(ci, carry):
            off = base + ci * ch
            pltpu.sync_copy(idx_hbm.at[pl.ds(off, ch)], idx_v)
            pltpu.sync_copy(opos_hbm.at[pl.ds(off, ch)], pos_v)
            pltpu.async_copy(table_hbm.at[idx_v], rows_v, sem).wait()
            pltpu.async_copy(rows_v, out_hbm.at[pos_v], sem).wait()
            return carry

        lax.fori_loop(0, nch, body, 0)

    return gather_k


@functools.lru_cache(maxsize=None)
def _sc_scatter_add(etot, ch):
    perw = etot // NW
    nch = perw // ch
    mesh = plsc.VectorSubcoreMesh(core_axis_name="c", subcore_axis_name="s")

    @functools.partial(
        pl.kernel, mesh=mesh,
        out_type=jax.ShapeDtypeStruct((2, N, 16), jnp.float32),
        compiler_params=pltpu.CompilerParams(use_tc_tiling_on_sc=False),
        scratch_types=[
            pltpu.VMEM((ch,), jnp.int32),
            pltpu.VMEM((ch,), jnp.int32),
            pltpu.VMEM((ch, 16), jnp.float32),
            pltpu.VMEM_SHARED((N, 16), jnp.float32),
            pltpu.SemaphoreType.DMA,
        ],
    )
    def scatter_k(vals_hbm, idx_hbm, vpos_hbm, zeros_hbm, out_hbm, idx_v,
                  pos_v, val_v, acc_sh, sem):
        cid = lax.axis_index("c")
        sid = lax.axis_index("s")

        @pl.when(sid == 0)
        def _init():
            pltpu.sync_copy(zeros_hbm, acc_sh)

        plsc.subcore_barrier()

        base = (sid * 2 + cid) * perw

        def body(ci, carry):
            off = base + ci * ch
            pltpu.sync_copy(idx_hbm.at[pl.ds(off, ch)], idx_v)
            pltpu.sync_copy(vpos_hbm.at[pl.ds(off, ch)], pos_v)
            pltpu.async_copy(vals_hbm.at[pos_v], val_v, sem).wait()
            pltpu.sync_copy(val_v, acc_sh.at[idx_v], add=True)
            return carry

        lax.fori_loop(0, nch, body, 0)
        plsc.subcore_barrier()

        rows = N // 16
        pltpu.sync_copy(acc_sh.at[pl.ds(sid * rows, rows)],
                        out_hbm.at[cid, pl.ds(sid * rows, rows)])

    return scatter_k


def _gather_a(*args):
    return _sc_gather(E_A, 1280)(*args)


def _gather_b(*args):
    return _sc_gather(E_B, 976)(*args)


def _scatter_a(*args):
    return _sc_scatter_add(E_A, 1280)(*args)


def _scatter_b(*args):
    return _sc_scatter_add(E_B, 976)(*args)


def _unpack(p):
    return jnp.concatenate([p[:, 16 * j:16 * (j + 1)] for j in range(8)],
                           axis=0)


def _pack(x):
    q = EB // 8
    return jnp.concatenate([x[q * j:q * (j + 1), :] for j in range(8)],
                           axis=1)


def _edges_math(ea, xj, w1_ref, b1_ref, w2_ref, b2_ref, t_ref, r_ref):
    h = jnp.dot(ea, w1_ref[...], preferred_element_type=jnp.float32)
    h = jnp.maximum(h + b1_ref[...], 0.0)
    wm = jnp.dot(h, w2_ref[...], preferred_element_type=jnp.float32)
    wm = wm + b2_ref[...]
    xt = jnp.dot(xj, t_ref[...], preferred_element_type=jnp.float32)
    return jnp.dot(xt * wm, r_ref[...], preferred_element_type=jnp.float32)


def _edges1_body(ea_ref, xj_ref, w1_ref, b1_ref, w2_ref, b2_ref, t_ref,
                 r_ref, out_ref, eap_ref):
    ea = ea_ref[...].reshape(EB, 16)
    xj = _unpack(xj_ref[...])
    msg = _edges_math(ea, xj, w1_ref, b1_ref, w2_ref, b2_ref, t_ref, r_ref)
    out_ref[...] = _pack(msg)
    eap_ref[...] = _pack(ea)


def _edges2_body(ea_ref, xj_ref, w1_ref, b1_ref, w2_ref, b2_ref, t_ref,
                 r_ref, out_ref):
    ea = _unpack(ea_ref[...])
    xj = _unpack(xj_ref[...])
    msg = _edges_math(ea, xj, w1_ref, b1_ref, w2_ref, b2_ref, t_ref, r_ref)
    out_ref[...] = _pack(msg)


def _weight_specs(h1, o1):
    return [
        pl.BlockSpec((16, h1), lambda i: (0, 0)),
        pl.BlockSpec((1, h1), lambda i: (0, 0)),
        pl.BlockSpec((h1, o1), lambda i: (0, 0)),
        pl.BlockSpec((1, o1), lambda i: (0, 0)),
        pl.BlockSpec((16, o1), lambda i: (0, 0)),
        pl.BlockSpec((o1, 16), lambda i: (0, 0)),
    ]


def _tc_edges1(ea_full, xj_p, w1, b1, w2, b2, t, r, blk0, ng):
    eh = ng * EB
    return pl.pallas_call(
        _edges1_body,
        grid=(ng,),
        in_specs=[
            pl.BlockSpec((EB // 8, 8, 16), lambda i: (i + blk0, 0, 0)),
            pl.BlockSpec((EB // 8, 128), lambda i: (i, 0)),
        ] + _weight_specs(w1.shape[1], w2.shape[1]),
        out_specs=[pl.BlockSpec((EB // 8, 128), lambda i: (i, 0)),
                   pl.BlockSpec((EB // 8, 128), lambda i: (i, 0))],
        out_shape=[jax.ShapeDtypeStruct((eh // 8, 128), jnp.float32),
                   jax.ShapeDtypeStruct((eh // 8, 128), jnp.float32)],
    )(ea_full, xj_p, w1, b1, w2, b2, t, r)


def _tc_edges2(ea_p, xj_p, w1, b1, w2, b2, t, r, ng):
    eh = ng * EB
    return pl.pallas_call(
        _edges2_body,
        grid=(ng,),
        in_specs=[
            pl.BlockSpec((EB // 8, 128), lambda i: (i, 0)),
            pl.BlockSpec((EB // 8, 128), lambda i: (i, 0)),
        ] + _weight_specs(w1.shape[1], w2.shape[1]),
        out_specs=pl.BlockSpec((EB // 8, 128), lambda i: (i, 0)),
        out_shape=jax.ShapeDtypeStruct((eh // 8, 128), jnp.float32),
    )(ea_p, xj_p, w1, b1, w2, b2, t, r)


def _comb1_body(pa_ref, pb_ref, x_ref, root_ref, bias_ref, out_ref):
    agg = pa_ref[0] + pa_ref[1] + pb_ref[0] + pb_ref[1]
    rt = jnp.dot(x_ref[...], root_ref[...], preferred_element_type=jnp.float32)
    out_ref[...] = jnp.maximum(agg + rt + bias_ref[...], 0.0)


def _tc_combine1(pa, pb, x, root, bias):
    return pl.pallas_call(
        _comb1_body,
        out_shape=jax.ShapeDtypeStruct((N, 16), jnp.float32),
    )(pa, pb, x, root, bias)


def _comb2_body(pa_ref, pb_ref, x_ref, root_ref, bias_ref, out_ref):
    agg = pa_ref[0] + pa_ref[1] + pb_ref[0] + pb_ref[1]
    rt = jnp.dot(x_ref[...], root_ref[...], preferred_element_type=jnp.float32)
    y = agg[:, :8] + rt + bias_ref[...]
    m = jnp.max(y, axis=1, keepdims=True)
    lse = jnp.log(jnp.sum(jnp.exp(y - m), axis=1, keepdims=True)) + m
    out_ref[...] = y - lse


def _tc_combine2(pa, pb, x, root, bias):
    return pl.pallas_call(
        _comb2_body,
        out_shape=jax.ShapeDtypeStruct((N, 8), jnp.float32),
    )(pa, pb, x, root, bias)


def _qperm(nblk):
    q = jnp.arange(nblk * EB, dtype=jnp.int32).reshape(nblk, EB // 8, 8)
    return q.transpose(0, 2, 1).reshape(nblk * EB)


def kernel(x_in, edge_index, edge_atts, w11, b11, w12, b12, root1, bias1,
           w21, b21, w22, b22, root2, bias2):
    src = edge_index[0]
    dst = edge_index[1]
    src_a, src_b = src[:E_A], src[E_A:]
    dst_a, dst_b = dst[:E_A], dst[E_A:]
    qa = _qperm(GA)
    qb = _qperm(GB)

    f32 = jnp.float32
    eye16 = jnp.eye(16, dtype=f32)
    t1 = jnp.repeat(eye16, 16, axis=1)
    r1 = jnp.tile(eye16, (16, 1))
    t2 = jnp.repeat(eye16, 8, axis=1)
    r2 = jnp.pad(jnp.tile(jnp.eye(8, dtype=f32), (16, 1)), ((0, 0), (0, 8)))

    zeros = jnp.zeros((N, 16), f32)
    ea_p = edge_atts.reshape(E // 8, 8, 16)
    b11r, b12r = b11[None, :], b12[None, :]
    b21r, b22r = b21[None, :], b22[None, :]

    xj1a = _gather_a(x_in, src_a, qa)
    xj1b = _gather_b(x_in, src_b, qb)
    msg1a, ear_a = _tc_edges1(ea_p, xj1a.reshape(E_A // 8, 128), w11, b11r,
                              w12, b12r, t1, r1, 0, GA)
    p1a = _scatter_a(msg1a.reshape(E_A, 16), dst_a, qa, zeros)
    msg1b, ear_b = _tc_edges1(ea_p, xj1b.reshape(E_B // 8, 128), w11, b11r,
                              w12, b12r, t1, r1, GA, GB)
    p1b = _scatter_b(msg1b.reshape(E_B, 16), dst_b, qb, zeros)
    x1 = _tc_combine1(p1a, p1b, x_in, root1, bias1[None, :])

    xj2a = _gather_a(x1, src_a, qa)
    xj2b = _gather_b(x1, src_b, qb)
    msg2a = _tc_edges2(ear_a, xj2a.reshape(E_A // 8, 128), w21, b21r,
                       w22, b22r, t2, r2, GA)
    p2a = _scatter_a(msg2a.reshape(E_A, 16), dst_a, qa, zeros)
    msg2b = _tc_edges2(ear_b, xj2b.reshape(E_B // 8, 128), w21, b21r,
                       w22, b22r, t2, r2, GB)
    p2b = _scatter_b(msg2b.reshape(E_B, 16), dst_b, qb, zeros)
    return _tc_combine2(p2a, p2b, x1, root2, bias2[None, :])
